# baseline (speedup 1.0000x reference)
import functools

import jax
import jax.numpy as jnp
from jax.experimental import pallas as pl
from jax.experimental.pallas import tpu as pltpu

T = 2048
D = 1024
H = 4096
E = 8

T_SPLIT = 2
T_BLK = T // T_SPLIT
HB = 512
H_TILES = H // HB


def _moe_body(x_ref, wr_ref, br_ref, w1_ref, b1_ref, w2_ref, b2_ref,
              out_ref, w_ref):
    e = pl.program_id(1)
    hb = pl.program_id(2)

    @pl.when((e == 0) & (hb == 0))
    def _init():
        logits = jnp.dot(x_ref[...], wr_ref[...],
                         preferred_element_type=jnp.float32)
        logits = logits + br_ref[0, :]
        w_ref[...] = jax.nn.softmax(logits, axis=-1)
        out_ref[...] = jnp.dot(w_ref[...], b2_ref[...],
                               preferred_element_type=jnp.float32)

    h = jnp.dot(x_ref[...], w1_ref[0], preferred_element_type=jnp.float32)
    h = h + b1_ref[e, pl.ds(hb * HB, HB)]
    h = jnp.maximum(h, 0.0)
    lane = jax.lax.broadcasted_iota(jnp.int32, (1, E), 1)
    wcol = jnp.sum(jnp.where(lane == e, w_ref[...], 0.0),
                   axis=1, keepdims=True)
    wh = (h * wcol).astype(jnp.bfloat16)
    out_ref[...] += jnp.dot(wh, w2_ref[0], preferred_element_type=jnp.float32)


@jax.jit
def kernel(x, Wr, br, W1, b1, W2, b2):
    xb = x.astype(jnp.bfloat16)
    wrb = Wr.astype(jnp.bfloat16)
    w1b = W1.astype(jnp.bfloat16)
    w2b = W2.astype(jnp.bfloat16)
    brr = br.reshape(1, E)

    grid = (T_SPLIT, E, H_TILES)
    return pl.pallas_call(
        _moe_body,
        grid=grid,
        in_specs=[
            pl.BlockSpec((T_BLK, D), lambda t, e, hb: (t, 0)),
            pl.BlockSpec((D, E), lambda t, e, hb: (0, 0)),
            pl.BlockSpec((1, E), lambda t, e, hb: (0, 0)),
            pl.BlockSpec((1, D, HB), lambda t, e, hb: (e, 0, hb)),
            pl.BlockSpec((E, H), lambda t, e, hb: (0, 0)),
            pl.BlockSpec((1, HB, D), lambda t, e, hb: (e, hb, 0)),
            pl.BlockSpec((E, D), lambda t, e, hb: (0, 0)),
        ],
        out_specs=pl.BlockSpec((T_BLK, D), lambda t, e, hb: (t, 0)),
        out_shape=jax.ShapeDtypeStruct((T, D), jnp.float32),
        scratch_shapes=[pltpu.VMEM((T_BLK, E), jnp.float32)],
        compiler_params=pltpu.CompilerParams(
            dimension_semantics=("parallel", "arbitrary", "arbitrary"),
        ),
    )(xb, wrb, brr, w1b, b1, w2b, b2)


# speedup vs baseline: 1.3594x; 1.3594x over previous
import functools

import jax
import jax.numpy as jnp
from jax.experimental import pallas as pl
from jax.experimental.pallas import tpu as pltpu

T = 2048
D = 1024
H = 4096
E = 8

T_SPLIT = 2
T_BLK = T // T_SPLIT
HB = 512
H_TILES = H // HB


def _moe_body(x_ref, wr_ref, br_ref, w1_ref, b1_ref, w2_ref, b2_ref,
              out_ref, w_ref):
    e = pl.program_id(1)
    hb = pl.program_id(2)

    @pl.when((e == 0) & (hb == 0))
    def _init():
        logits = jnp.dot(x_ref[...], wr_ref[...],
                         preferred_element_type=jnp.float32)
        logits = logits + br_ref[0, :]
        w_ref[...] = jax.nn.softmax(logits, axis=-1)
        out_ref[...] = jnp.dot(w_ref[...], b2_ref[...],
                               preferred_element_type=jnp.float32)

    h = jnp.dot(x_ref[...], w1_ref[0].astype(jnp.bfloat16),
                preferred_element_type=jnp.float32)
    h = h + b1_ref[e, pl.ds(hb * HB, HB)]
    h = jnp.maximum(h, 0.0)
    lane = jax.lax.broadcasted_iota(jnp.int32, (1, E), 1)
    wcol = jnp.sum(jnp.where(lane == e, w_ref[...], 0.0),
                   axis=1, keepdims=True)
    wh = (h * wcol).astype(jnp.bfloat16)
    out_ref[...] += jnp.dot(wh, w2_ref[0].astype(jnp.bfloat16),
                            preferred_element_type=jnp.float32)


@jax.jit
def kernel(x, Wr, br, W1, b1, W2, b2):
    xb = x.astype(jnp.bfloat16)
    wrb = Wr.astype(jnp.bfloat16)
    brr = br.reshape(1, E)

    grid = (T_SPLIT, E, H_TILES)
    return pl.pallas_call(
        _moe_body,
        grid=grid,
        in_specs=[
            pl.BlockSpec((T_BLK, D), lambda t, e, hb: (t, 0)),
            pl.BlockSpec((D, E), lambda t, e, hb: (0, 0)),
            pl.BlockSpec((1, E), lambda t, e, hb: (0, 0)),
            pl.BlockSpec((1, D, HB), lambda t, e, hb: (e, 0, hb)),
            pl.BlockSpec((E, H), lambda t, e, hb: (0, 0)),
            pl.BlockSpec((1, HB, D), lambda t, e, hb: (e, hb, 0)),
            pl.BlockSpec((E, D), lambda t, e, hb: (0, 0)),
        ],
        out_specs=pl.BlockSpec((T_BLK, D), lambda t, e, hb: (t, 0)),
        out_shape=jax.ShapeDtypeStruct((T, D), jnp.float32),
        scratch_shapes=[pltpu.VMEM((T_BLK, E), jnp.float32)],
        compiler_params=pltpu.CompilerParams(
            dimension_semantics=("parallel", "arbitrary", "arbitrary"),
        ),
    )(xb, wrb, brr, W1, b1, W2, b2)


# speedup vs baseline: 1.4644x; 1.0772x over previous
import functools

import jax
import jax.numpy as jnp
from jax.experimental import pallas as pl
from jax.experimental.pallas import tpu as pltpu

T = 2048
D = 1024
H = 4096
E = 8

T_SPLIT = 2
T_BLK = T // T_SPLIT
HB = 1024
H_TILES = H // HB


def _moe_body(x_ref, wr_ref, br_ref, w1_ref, b1_ref, w2_ref, b2_ref,
              out_ref, w_ref):
    e = pl.program_id(1)
    hb = pl.program_id(2)

    @pl.when((e == 0) & (hb == 0))
    def _init():
        logits = jnp.dot(x_ref[...], wr_ref[...],
                         preferred_element_type=jnp.float32)
        logits = logits + br_ref[0, :]
        w_ref[...] = jax.nn.softmax(logits, axis=-1)
        out_ref[...] = jnp.dot(w_ref[...], b2_ref[...],
                               preferred_element_type=jnp.float32)

    h = jnp.dot(x_ref[...], w1_ref[0].astype(jnp.bfloat16),
                preferred_element_type=jnp.float32)
    h = h + b1_ref[e, pl.ds(hb * HB, HB)]
    h = jnp.maximum(h, 0.0)
    lane = jax.lax.broadcasted_iota(jnp.int32, (1, E), 1)
    wcol = jnp.sum(jnp.where(lane == e, w_ref[...], 0.0),
                   axis=1, keepdims=True)
    wh = (h * wcol).astype(jnp.bfloat16)
    out_ref[...] += jnp.dot(wh, w2_ref[0].astype(jnp.bfloat16),
                            preferred_element_type=jnp.float32)


@jax.jit
def kernel(x, Wr, br, W1, b1, W2, b2):
    xb = x.astype(jnp.bfloat16)
    wrb = Wr.astype(jnp.bfloat16)
    brr = br.reshape(1, E)

    grid = (T_SPLIT, E, H_TILES)
    return pl.pallas_call(
        _moe_body,
        grid=grid,
        in_specs=[
            pl.BlockSpec((T_BLK, D), lambda t, e, hb: (t, 0)),
            pl.BlockSpec((D, E), lambda t, e, hb: (0, 0)),
            pl.BlockSpec((1, E), lambda t, e, hb: (0, 0)),
            pl.BlockSpec((1, D, HB), lambda t, e, hb: (e, 0, hb)),
            pl.BlockSpec((E, H), lambda t, e, hb: (0, 0)),
            pl.BlockSpec((1, HB, D), lambda t, e, hb: (e, hb, 0)),
            pl.BlockSpec((E, D), lambda t, e, hb: (0, 0)),
        ],
        out_specs=pl.BlockSpec((T_BLK, D), lambda t, e, hb: (t, 0)),
        out_shape=jax.ShapeDtypeStruct((T, D), jnp.float32),
        scratch_shapes=[pltpu.VMEM((T_BLK, E), jnp.float32)],
        compiler_params=pltpu.CompilerParams(
            dimension_semantics=("parallel", "arbitrary", "arbitrary"),
        ),
    )(xb, wrb, brr, W1, b1, W2, b2)


# speedup vs baseline: 1.5095x; 1.0308x over previous
import functools

import jax
import jax.numpy as jnp
from jax.experimental import pallas as pl
from jax.experimental.pallas import tpu as pltpu

T = 2048
D = 1024
H = 4096
E = 8

T_SPLIT = 2
T_BLK = T // T_SPLIT
HB = 2048
H_TILES = H // HB


def _moe_body(x_ref, wr_ref, br_ref, w1_ref, b1_ref, w2_ref, b2_ref,
              out_ref, w_ref):
    e = pl.program_id(1)
    hb = pl.program_id(2)

    @pl.when((e == 0) & (hb == 0))
    def _init():
        logits = jnp.dot(x_ref[...], wr_ref[...],
                         preferred_element_type=jnp.float32)
        logits = logits + br_ref[0, :]
        w_ref[...] = jax.nn.softmax(logits, axis=-1)
        out_ref[...] = jnp.dot(w_ref[...], b2_ref[...],
                               preferred_element_type=jnp.float32)

    h = jnp.dot(x_ref[...], w1_ref[0].astype(jnp.bfloat16),
                preferred_element_type=jnp.float32)
    h = h + b1_ref[e, pl.ds(hb * HB, HB)]
    h = jnp.maximum(h, 0.0)
    lane = jax.lax.broadcasted_iota(jnp.int32, (1, E), 1)
    wcol = jnp.sum(jnp.where(lane == e, w_ref[...], 0.0),
                   axis=1, keepdims=True)
    wh = (h * wcol).astype(jnp.bfloat16)
    out_ref[...] += jnp.dot(wh, w2_ref[0].astype(jnp.bfloat16),
                            preferred_element_type=jnp.float32)


@jax.jit
def kernel(x, Wr, br, W1, b1, W2, b2):
    xb = x.astype(jnp.bfloat16)
    wrb = Wr.astype(jnp.bfloat16)
    brr = br.reshape(1, E)

    grid = (T_SPLIT, E, H_TILES)
    return pl.pallas_call(
        _moe_body,
        grid=grid,
        in_specs=[
            pl.BlockSpec((T_BLK, D), lambda t, e, hb: (t, 0)),
            pl.BlockSpec((D, E), lambda t, e, hb: (0, 0)),
            pl.BlockSpec((1, E), lambda t, e, hb: (0, 0)),
            pl.BlockSpec((1, D, HB), lambda t, e, hb: (e, 0, hb)),
            pl.BlockSpec((E, H), lambda t, e, hb: (0, 0)),
            pl.BlockSpec((1, HB, D), lambda t, e, hb: (e, hb, 0)),
            pl.BlockSpec((E, D), lambda t, e, hb: (0, 0)),
        ],
        out_specs=pl.BlockSpec((T_BLK, D), lambda t, e, hb: (t, 0)),
        out_shape=jax.ShapeDtypeStruct((T, D), jnp.float32),
        scratch_shapes=[pltpu.VMEM((T_BLK, E), jnp.float32)],
        compiler_params=pltpu.CompilerParams(
            dimension_semantics=("parallel", "arbitrary", "arbitrary"),
        ),
    )(xb, wrb, brr, W1, b1, W2, b2)


# speedup vs baseline: 1.5150x; 1.0037x over previous
import functools

import jax
import jax.numpy as jnp
from jax.experimental import pallas as pl
from jax.experimental.pallas import tpu as pltpu

T = 2048
D = 1024
H = 4096
E = 8

T_SPLIT = 2
T_BLK = T // T_SPLIT
HB = 2048
H_TILES = H // HB


def _moe_body(x_ref, wr_ref, br_ref, w1_ref, b1_ref, w2_ref, b2_ref,
              out_ref, w_ref):
    e = pl.program_id(1)
    hb = pl.program_id(2)

    @pl.when((e == 0) & (hb == 0))
    def _init():
        logits = jnp.dot(x_ref[...], wr_ref[...],
                         preferred_element_type=jnp.float32)
        logits = logits + br_ref[0, :]
        w_ref[...] = jax.nn.softmax(logits, axis=-1)
        out_ref[...] = jnp.dot(w_ref[...], b2_ref[...],
                               preferred_element_type=jnp.float32)

    h = jnp.dot(x_ref[...], w1_ref[0].astype(jnp.bfloat16),
                preferred_element_type=jnp.float32)
    h = h + b1_ref[e, pl.ds(hb * HB, HB)]
    h = jnp.maximum(h, 0.0).astype(jnp.bfloat16)
    lane = jax.lax.broadcasted_iota(jnp.int32, (1, E), 1)
    wcol = jnp.sum(jnp.where(lane == e, w_ref[...], 0.0),
                   axis=1, keepdims=True)
    wh = h * wcol.astype(jnp.bfloat16)
    out_ref[...] += jnp.dot(wh, w2_ref[0].astype(jnp.bfloat16),
                            preferred_element_type=jnp.float32)


@jax.jit
def kernel(x, Wr, br, W1, b1, W2, b2):
    xb = x.astype(jnp.bfloat16)
    wrb = Wr.astype(jnp.bfloat16)
    brr = br.reshape(1, E)

    grid = (T_SPLIT, E, H_TILES)
    return pl.pallas_call(
        _moe_body,
        grid=grid,
        in_specs=[
            pl.BlockSpec((T_BLK, D), lambda t, e, hb: (t, 0)),
            pl.BlockSpec((D, E), lambda t, e, hb: (0, 0)),
            pl.BlockSpec((1, E), lambda t, e, hb: (0, 0)),
            pl.BlockSpec((1, D, HB), lambda t, e, hb: (e, 0, hb)),
            pl.BlockSpec((E, H), lambda t, e, hb: (0, 0)),
            pl.BlockSpec((1, HB, D), lambda t, e, hb: (e, hb, 0)),
            pl.BlockSpec((E, D), lambda t, e, hb: (0, 0)),
        ],
        out_specs=pl.BlockSpec((T_BLK, D), lambda t, e, hb: (t, 0)),
        out_shape=jax.ShapeDtypeStruct((T, D), jnp.float32),
        scratch_shapes=[pltpu.VMEM((T_BLK, E), jnp.float32)],
        compiler_params=pltpu.CompilerParams(
            dimension_semantics=("parallel", "arbitrary", "arbitrary"),
        ),
    )(xb, wrb, brr, W1, b1, W2, b2)
